# baseline (speedup 1.0000x reference)
import functools

import jax
import jax.numpy as jnp
from jax import lax
from jax.experimental import pallas as pl
from jax.experimental.pallas import tpu as pltpu
from jax.experimental.pallas import tpu_sc as plsc

NPINS = 1_600_000
NNETS = 100_000
NC = 2
NS = 16
NW = NC * NS
NPW = 3136
NLOC = 3152
DUMMY = NLOC - 1
C = 2048
NPAD = NW * NPW


def _take(v, idx):
    return v.at[idx].get(mode="promise_in_bounds")


def _extract(vmem, j, iota):
    del iota
    base = pl.multiple_of((j // 16) * 16, 16)
    v = vmem[pl.ds(base, 16)]
    sel = _take(v, jnp.broadcast_to(j - base, (16,)))
    return sel[0]


def _sc_body(pos_h, p2n_h, bnd_h, wx_h, wy_h, gi_h, out_h,
             bnds_v, gi_v, wx_v, wy_v, nbuf, xbuf, ybuf,
             mxx, mnx, mxy, mny,
             spx, sxpx, snx, sxnx, spy, sxpy, sny, sxny, accv):
    iota = lax.broadcasted_iota(jnp.int32, (16,), 0)
    c = lax.axis_index("c")
    s = lax.axis_index("s")
    wid = s * NC + c
    n0 = pl.multiple_of(wid * NPW, 16)
    ncnt = jnp.minimum(NNETS - n0, NPW)

    pltpu.sync_copy(bnd_h, bnds_v)
    pltpu.sync_copy(gi_h, gi_v)
    pltpu.sync_copy(wx_h.at[pl.ds(n0, NPW)], wx_v)
    pltpu.sync_copy(wy_h.at[pl.ds(n0, NPW)], wy_v)

    p0 = _extract(bnds_v, wid, iota)
    p1 = _extract(bnds_v, wid + 1, iota)
    a0 = pl.multiple_of((p0 // 16) * 16, 16)
    nch = (p1 - a0 + C - 1) // C

    gv = gi_v[...]

    big = jnp.full((16,), 3.0e38, jnp.float32)
    zf = jnp.zeros((16,), jnp.float32)

    def initb(j, carry):
        dsl = pl.ds(pl.multiple_of(j * 16, 16), 16)
        mxx[dsl] = -big
        mnx[dsl] = big
        mxy[dsl] = -big
        mny[dsl] = big
        spx[dsl] = zf
        sxpx[dsl] = zf
        snx[dsl] = zf
        sxnx[dsl] = zf
        spy[dsl] = zf
        sxpy[dsl] = zf
        sny[dsl] = zf
        sxny[dsl] = zf
        return carry

    lax.fori_loop(0, NLOC // 16, initb, 0)
    accv[...] = zf

    def load_chunk(i):
        bu = a0 + i * C
        base = pl.multiple_of(jnp.minimum(bu, NPINS - C), 16)
        lo = jnp.maximum(p0, bu)
        pltpu.sync_copy(p2n_h.at[pl.ds(base, C)], nbuf)
        pltpu.sync_copy(pos_h.at[pl.ds(base, C)], xbuf)
        pltpu.sync_copy(pos_h.at[pl.ds(base + NPINS, C)], ybuf)
        return base, lo

    idxk = tuple(jnp.maximum(iota - k, 0) for k in (1, 2, 4, 8))
    idxn = jnp.minimum(iota + 1, 15)

    def group_prelude(base, lo, off):
        nv = nbuf[pl.ds(off, 16)]
        xv = xbuf[pl.ds(off, 16)]
        yv = ybuf[pl.ds(off, 16)]
        g = base + off + iota
        mval = (g >= lo) & (g < p1)
        nloc = jnp.where(mval, nv - n0, DUMMY)
        nprev = _take(nloc, idxk[0])
        isf = (iota == 0) | (nloc != nprev)
        hd = jnp.where(isf, iota, 0)
        for kk in range(4):
            hd = jnp.maximum(hd, _take(hd, idxk[kk]))
        sks = tuple((iota - k) >= hd for k in (1, 2, 4, 8))
        nnxt = _take(nloc, idxn)
        isl = (iota == 15) | (nloc != nnxt)
        lm = isl & mval
        return nv, xv, yv, mval, nloc, sks, lm

    def chunk1(i, carry):
        base, lo = load_chunk(i)

        def grp(j, carry2):
            off = pl.multiple_of(j * 16, 16)
            _, xv, yv, mval, nloc, sks, lm = group_prelude(base, lo, off)

            def segscan(v, op):
                m = v
                for kk in range(4):
                    m = jnp.where(sks[kk], op(_take(m, idxk[kk]), m), m)
                return m

            sxmax = segscan(xv, jnp.maximum)
            sxmin = segscan(xv, jnp.minimum)
            symax = segscan(yv, jnp.maximum)
            symin = segscan(yv, jnp.minimum)
            cur = plsc.load_gather(mxx, [nloc])
            plsc.store_scatter(mxx, [nloc], jnp.maximum(cur, sxmax), mask=lm)
            cur = plsc.load_gather(mnx, [nloc])
            plsc.store_scatter(mnx, [nloc], jnp.minimum(cur, sxmin), mask=lm)
            cur = plsc.load_gather(mxy, [nloc])
            plsc.store_scatter(mxy, [nloc], jnp.maximum(cur, symax), mask=lm)
            cur = plsc.load_gather(mny, [nloc])
            plsc.store_scatter(mny, [nloc], jnp.minimum(cur, symin), mask=lm)
            return carry2

        lax.fori_loop(0, C // 16, grp, 0)
        return carry

    lax.fori_loop(0, nch, chunk1, 0)

    def chunk2(i, carry):
        base, lo = load_chunk(i)

        def grp(j, carry2):
            off = pl.multiple_of(j * 16, 16)
            nv = nbuf[pl.ds(off, 16)]
            xv = xbuf[pl.ds(off, 16)]
            yv = ybuf[pl.ds(off, 16)]
            g = base + off + iota
            mval = (g >= lo) & (g < p1)
            nloc = jnp.where(mval, nv - n0, DUMMY)
            mpx = plsc.load_gather(mxx, [nloc])
            mnx_ = plsc.load_gather(mnx, [nloc])
            mpy = plsc.load_gather(mxy, [nloc])
            mny_ = plsc.load_gather(mny, [nloc])
            epx = jnp.where(mval, jnp.exp((xv - mpx) * gv), 0.0)
            enx = jnp.where(mval, jnp.exp((mnx_ - xv) * gv), 0.0)
            epy = jnp.where(mval, jnp.exp((yv - mpy) * gv), 0.0)
            eny = jnp.where(mval, jnp.exp((mny_ - yv) * gv), 0.0)
            plsc.addupdate_scatter(spx, [nloc], epx, mask=mval)
            plsc.addupdate_scatter(sxpx, [nloc], xv * epx, mask=mval)
            plsc.addupdate_scatter(snx, [nloc], enx, mask=mval)
            plsc.addupdate_scatter(sxnx, [nloc], xv * enx, mask=mval)
            plsc.addupdate_scatter(spy, [nloc], epy, mask=mval)
            plsc.addupdate_scatter(sxpy, [nloc], yv * epy, mask=mval)
            plsc.addupdate_scatter(sny, [nloc], eny, mask=mval)
            plsc.addupdate_scatter(sxny, [nloc], yv * eny, mask=mval)
            return carry2

        lax.fori_loop(0, C // 16, grp, 0)
        return carry

    lax.fori_loop(0, nch, chunk2, 0)

    one = jnp.ones((16,), jnp.float32)

    def net_grp(j, carry):
        dsl = pl.ds(pl.multiple_of(j * 16, 16), 16)
        ok = (j * 16 + iota) < ncnt
        sp = spx[dsl]
        has = ok & (sp > 0.0)
        spd = jnp.where(has, sp, one)
        snd = jnp.where(has, snx[dsl], one)
        wlx = sxpx[dsl] / spd - sxnx[dsl] / snd
        spd2 = jnp.where(has, spy[dsl], one)
        snd2 = jnp.where(has, sny[dsl], one)
        wly = sxpy[dsl] / spd2 - sxny[dsl] / snd2
        contrib = jnp.where(has, wx_v[dsl] * wlx + wy_v[dsl] * wly, 0.0)
        accv[...] = accv[...] + contrib
        return carry

    lax.fori_loop(0, NPW // 16, net_grp, 0)
    pltpu.sync_copy(accv, out_h.at[wid])


@jax.jit
def _wirelength_sc(pos, p2n, bounds, wx, wy, ginv):
    mesh = plsc.VectorSubcoreMesh(core_axis_name="c", subcore_axis_name="s")
    f = functools.partial(
        pl.kernel,
        mesh=mesh,
        compiler_params=pltpu.CompilerParams(needs_layout_passes=False),
        out_type=jax.ShapeDtypeStruct((NW, 16), jnp.float32),
        scratch_types=[
            pltpu.VMEM((48,), jnp.int32),
            pltpu.VMEM((16,), jnp.float32),
            pltpu.VMEM((NPW,), jnp.float32),
            pltpu.VMEM((NPW,), jnp.float32),
            pltpu.VMEM((C,), jnp.int32),
            pltpu.VMEM((C,), jnp.float32),
            pltpu.VMEM((C,), jnp.float32),
        ] + [pltpu.VMEM((NLOC,), jnp.float32)] * 12 + [
            pltpu.VMEM((16,), jnp.float32),
        ],
    )(_sc_body)
    return f(pos, p2n, bounds, wx, wy, ginv)


def kernel(pos, pin2net_map, flat_netpin, netpin_start, net_weights,
           net_weights_x, net_mask, pin_mask, inv_gamma):
    maskf = net_mask.astype(jnp.float32)
    wx = jnp.pad(net_weights_x * maskf, (0, NPAD - NNETS))
    wy = jnp.pad(net_weights * maskf, (0, NPAD - NNETS))
    bidx = jnp.minimum(jnp.arange(33, dtype=jnp.int32) * NPW, NNETS)
    bounds = jnp.pad(netpin_start[bidx], (0, 48 - 33), mode="edge")
    ginv = jnp.broadcast_to(inv_gamma.astype(jnp.float32), (16,))
    out = _wirelength_sc(pos, pin2net_map, bounds, wx, wy, ginv)
    return jnp.sum(out)


# speedup vs baseline: 2.0912x; 2.0912x over previous
import functools

import jax
import jax.numpy as jnp
from jax import lax
from jax.experimental import pallas as pl
from jax.experimental.pallas import tpu as pltpu
from jax.experimental.pallas import tpu_sc as plsc

NPINS = 1_600_000
NNETS = 100_000
NC = 2
NS = 16
NW = NC * NS
NPW = 3136
NLOC = 3152
DUMMY = NLOC - 1
C = 4096
NPAD = NW * NPW


def _take(v, idx):
    return v.at[idx].get(mode="promise_in_bounds")


def _extract(vmem, j, iota):
    del iota
    base = pl.multiple_of((j // 16) * 16, 16)
    v = vmem[pl.ds(base, 16)]
    sel = _take(v, jnp.broadcast_to(j - base, (16,)))
    return sel[0]


def _sc_body(pos_h, p2n_h, bnd_h, wx_h, wy_h, gi_h, out_h,
             bnds_v, gi_v, wx_v, wy_v, nbuf, xbuf, ybuf,
             mxx, mnx, mxy, mny,
             spx, sxpx, snx, sxnx, spy, sxpy, sny, sxny, accv, sem_a, sem_b):
    iota = lax.broadcasted_iota(jnp.int32, (16,), 0)
    c = lax.axis_index("c")
    s = lax.axis_index("s")
    wid = s * NC + c
    n0 = pl.multiple_of(wid * NPW, 16)
    ncnt = jnp.minimum(NNETS - n0, NPW)

    pltpu.sync_copy(bnd_h, bnds_v)
    pltpu.sync_copy(gi_h, gi_v)
    pltpu.sync_copy(wx_h.at[pl.ds(n0, NPW)], wx_v)
    pltpu.sync_copy(wy_h.at[pl.ds(n0, NPW)], wy_v)

    p0 = _extract(bnds_v, wid, iota)
    p1 = _extract(bnds_v, wid + 1, iota)
    a0 = pl.multiple_of((p0 // 16) * 16, 16)
    nch = (p1 - a0 + C - 1) // C

    gv = gi_v[...]

    big = jnp.full((16,), 3.0e38, jnp.float32)
    zf = jnp.zeros((16,), jnp.float32)

    def initb(j, carry):
        dsl = pl.ds(pl.multiple_of(j * 16, 16), 16)
        mxx[dsl] = -big
        mnx[dsl] = big
        mxy[dsl] = -big
        mny[dsl] = big
        spx[dsl] = zf
        sxpx[dsl] = zf
        snx[dsl] = zf
        sxnx[dsl] = zf
        spy[dsl] = zf
        sxpy[dsl] = zf
        sny[dsl] = zf
        sxny[dsl] = zf
        return carry

    lax.fori_loop(0, NLOC // 16, initb, 0)
    accv[...] = zf

    idxk = tuple(jnp.maximum(iota - k, 0) for k in (1, 2, 4, 8))
    idxn = jnp.minimum(iota + 1, 15)

    def issue(i, boff, sem):
        bu = a0 + i * C
        base = pl.multiple_of(jnp.minimum(bu, NPINS - C), 16)
        pltpu.make_async_copy(
            p2n_h.at[pl.ds(base, C)], nbuf.at[pl.ds(boff, C)], sem).start()
        pltpu.make_async_copy(
            pos_h.at[pl.ds(base, C)], xbuf.at[pl.ds(boff, C)], sem).start()
        pltpu.make_async_copy(
            pos_h.at[pl.ds(base + NPINS, C)], ybuf.at[pl.ds(boff, C)],
            sem).start()

    def drain(boff, sem):
        pltpu.make_async_copy(
            p2n_h.at[pl.ds(0, C)], nbuf.at[pl.ds(boff, C)], sem).wait()
        pltpu.make_async_copy(
            pos_h.at[pl.ds(0, C)], xbuf.at[pl.ds(boff, C)], sem).wait()
        pltpu.make_async_copy(
            pos_h.at[pl.ds(0, C)], ybuf.at[pl.ds(boff, C)], sem).wait()

    def do_pass(grp_fn):
        def proc(i, boff):
            bu = a0 + i * C
            base = pl.multiple_of(jnp.minimum(bu, NPINS - C), 16)
            lo = jnp.maximum(p0, bu)

            def grp(j, carry2):
                boff16 = pl.multiple_of(boff + j * 16, 16)
                nv = nbuf[pl.ds(boff16, 16)]
                xv = xbuf[pl.ds(boff16, 16)]
                yv = ybuf[pl.ds(boff16, 16)]
                g = base + j * 16 + iota
                mval = (g >= lo) & (g < p1)
                nloc = jnp.where(mval, nv - n0, DUMMY)
                grp_fn(xv, yv, mval, nloc)
                return carry2

            lax.fori_loop(0, C // 16, grp, 0)

        pl.when(nch > 0)(lambda: issue(0, 0, sem_a))

        def body2(i2, carry):
            i = i2 * 2
            pl.when(i + 1 < nch)(lambda: issue(i + 1, C, sem_b))
            drain(0, sem_a)
            proc(i, 0)

            @pl.when(i + 1 < nch)
            def _():
                pl.when(i + 2 < nch)(lambda: issue(i + 2, 0, sem_a))
                drain(C, sem_b)
                proc(i + 1, C)

            return carry

        lax.fori_loop(0, (nch + 1) // 2, body2, 0)

    def seg_masks(nloc):
        nprev = _take(nloc, idxk[0])
        isf = (iota == 0) | (nloc != nprev)
        hd = jnp.where(isf, iota, 0)
        for kk in range(4):
            hd = jnp.maximum(hd, _take(hd, idxk[kk]))
        sks = tuple((iota - k) >= hd for k in (1, 2, 4, 8))
        nnxt = _take(nloc, idxn)
        isl = (iota == 15) | (nloc != nnxt)
        return sks, isl

    def grp1(xv, yv, mval, nloc):
        sks, isl = seg_masks(nloc)
        lm = isl & mval

        def segscan(v, op):
            m = v
            for kk in range(4):
                m = jnp.where(sks[kk], op(_take(m, idxk[kk]), m), m)
            return m

        cur = plsc.load_gather(mxx, [nloc])
        plsc.store_scatter(mxx, [nloc],
                           jnp.maximum(cur, segscan(xv, jnp.maximum)), mask=lm)
        cur = plsc.load_gather(mnx, [nloc])
        plsc.store_scatter(mnx, [nloc],
                           jnp.minimum(cur, segscan(xv, jnp.minimum)), mask=lm)
        cur = plsc.load_gather(mxy, [nloc])
        plsc.store_scatter(mxy, [nloc],
                           jnp.maximum(cur, segscan(yv, jnp.maximum)), mask=lm)
        cur = plsc.load_gather(mny, [nloc])
        plsc.store_scatter(mny, [nloc],
                           jnp.minimum(cur, segscan(yv, jnp.minimum)), mask=lm)

    def grp2(xv, yv, mval, nloc):
        sks, isl = seg_masks(nloc)
        lm = isl & mval
        mpx = plsc.load_gather(mxx, [nloc])
        mnx_ = plsc.load_gather(mnx, [nloc])
        mpy = plsc.load_gather(mxy, [nloc])
        mny_ = plsc.load_gather(mny, [nloc])
        epx = jnp.where(mval, jnp.exp((xv - mpx) * gv), 0.0)
        enx = jnp.where(mval, jnp.exp((mnx_ - xv) * gv), 0.0)
        epy = jnp.where(mval, jnp.exp((yv - mpy) * gv), 0.0)
        eny = jnp.where(mval, jnp.exp((mny_ - yv) * gv), 0.0)

        def runsum(q):
            m = q
            for kk in range(4):
                m = jnp.where(sks[kk], _take(m, idxk[kk]) + m, m)
            return m

        plsc.addupdate_scatter(spx, [nloc], runsum(epx), mask=lm)
        plsc.addupdate_scatter(sxpx, [nloc], runsum(xv * epx), mask=lm)
        plsc.addupdate_scatter(snx, [nloc], runsum(enx), mask=lm)
        plsc.addupdate_scatter(sxnx, [nloc], runsum(xv * enx), mask=lm)
        plsc.addupdate_scatter(spy, [nloc], runsum(epy), mask=lm)
        plsc.addupdate_scatter(sxpy, [nloc], runsum(yv * epy), mask=lm)
        plsc.addupdate_scatter(sny, [nloc], runsum(eny), mask=lm)
        plsc.addupdate_scatter(sxny, [nloc], runsum(yv * eny), mask=lm)

    do_pass(grp1)
    do_pass(grp2)

    one = jnp.ones((16,), jnp.float32)

    def net_grp(j, carry):
        dsl = pl.ds(pl.multiple_of(j * 16, 16), 16)
        ok = (j * 16 + iota) < ncnt
        sp = spx[dsl]
        has = ok & (sp > 0.0)
        spd = jnp.where(has, sp, one)
        snd = jnp.where(has, snx[dsl], one)
        wlx = sxpx[dsl] / spd - sxnx[dsl] / snd
        spd2 = jnp.where(has, spy[dsl], one)
        snd2 = jnp.where(has, sny[dsl], one)
        wly = sxpy[dsl] / spd2 - sxny[dsl] / snd2
        contrib = jnp.where(has, wx_v[dsl] * wlx + wy_v[dsl] * wly, 0.0)
        accv[...] = accv[...] + contrib
        return carry

    lax.fori_loop(0, NPW // 16, net_grp, 0)
    pltpu.sync_copy(accv, out_h.at[wid])


@jax.jit
def _wirelength_sc(pos, p2n, bounds, wx, wy, ginv):
    mesh = plsc.VectorSubcoreMesh(core_axis_name="c", subcore_axis_name="s")
    f = functools.partial(
        pl.kernel,
        mesh=mesh,
        compiler_params=pltpu.CompilerParams(needs_layout_passes=False),
        out_type=jax.ShapeDtypeStruct((NW, 16), jnp.float32),
        scratch_types=[
            pltpu.VMEM((48,), jnp.int32),
            pltpu.VMEM((16,), jnp.float32),
            pltpu.VMEM((NPW,), jnp.float32),
            pltpu.VMEM((NPW,), jnp.float32),
            pltpu.VMEM((2 * C,), jnp.int32),
            pltpu.VMEM((2 * C,), jnp.float32),
            pltpu.VMEM((2 * C,), jnp.float32),
        ] + [pltpu.VMEM((NLOC,), jnp.float32)] * 12 + [
            pltpu.VMEM((16,), jnp.float32),
            pltpu.SemaphoreType.DMA,
            pltpu.SemaphoreType.DMA,
        ],
    )(_sc_body)
    return f(pos, p2n, bounds, wx, wy, ginv)


def kernel(pos, pin2net_map, flat_netpin, netpin_start, net_weights,
           net_weights_x, net_mask, pin_mask, inv_gamma):
    maskf = net_mask.astype(jnp.float32)
    wx = jnp.pad(net_weights_x * maskf, (0, NPAD - NNETS))
    wy = jnp.pad(net_weights * maskf, (0, NPAD - NNETS))
    bidx = jnp.minimum(jnp.arange(33, dtype=jnp.int32) * NPW, NNETS)
    bounds = jnp.pad(netpin_start[bidx], (0, 48 - 33), mode="edge")
    ginv = jnp.broadcast_to(inv_gamma.astype(jnp.float32), (16,))
    out = _wirelength_sc(pos, pin2net_map, bounds, wx, wy, ginv)
    return jnp.sum(out)


# speedup vs baseline: 2.1730x; 1.0392x over previous
import functools

import jax
import jax.numpy as jnp
from jax import lax
from jax.experimental import pallas as pl
from jax.experimental.pallas import tpu as pltpu
from jax.experimental.pallas import tpu_sc as plsc

NPINS = 1_600_000
NNETS = 100_000
NC = 2
NS = 16
NW = NC * NS
NPW = 3136
NLOC = 3152
DUMMY = NLOC - 1
C = 8192
NPAD = NW * NPW


def _take(v, idx):
    return v.at[idx].get(mode="promise_in_bounds")


def _extract(vmem, j, iota):
    del iota
    base = pl.multiple_of((j // 16) * 16, 16)
    v = vmem[pl.ds(base, 16)]
    sel = _take(v, jnp.broadcast_to(j - base, (16,)))
    return sel[0]


def _sc_body(pos_h, p2n_h, bnd_h, wx_h, wy_h, gi_h, out_h,
             bnds_v, gi_v, wx_v, wy_v, nbuf, xbuf, ybuf,
             mxx, mnx, mxy, mny,
             spx, sxpx, snx, sxnx, spy, sxpy, sny, sxny, accv, sem_a, sem_b):
    iota = lax.broadcasted_iota(jnp.int32, (16,), 0)
    c = lax.axis_index("c")
    s = lax.axis_index("s")
    wid = s * NC + c
    n0 = pl.multiple_of(wid * NPW, 16)
    ncnt = jnp.minimum(NNETS - n0, NPW)

    pltpu.sync_copy(bnd_h, bnds_v)
    pltpu.sync_copy(gi_h, gi_v)
    pltpu.sync_copy(wx_h.at[pl.ds(n0, NPW)], wx_v)
    pltpu.sync_copy(wy_h.at[pl.ds(n0, NPW)], wy_v)

    p0 = _extract(bnds_v, wid, iota)
    p1 = _extract(bnds_v, wid + 1, iota)
    a0 = pl.multiple_of((p0 // 16) * 16, 16)
    nch = (p1 - a0 + C - 1) // C

    gv = gi_v[...]

    big = jnp.full((16,), 3.0e38, jnp.float32)
    zf = jnp.zeros((16,), jnp.float32)

    def initb(j, carry):
        dsl = pl.ds(pl.multiple_of(j * 16, 16), 16)
        mxx[dsl] = -big
        mnx[dsl] = big
        mxy[dsl] = -big
        mny[dsl] = big
        spx[dsl] = zf
        sxpx[dsl] = zf
        snx[dsl] = zf
        sxnx[dsl] = zf
        spy[dsl] = zf
        sxpy[dsl] = zf
        sny[dsl] = zf
        sxny[dsl] = zf
        return carry

    lax.fori_loop(0, NLOC // 16, initb, 0)
    accv[...] = zf

    idxk = tuple(jnp.maximum(iota - k, 0) for k in (1, 2, 4, 8))
    idxn = jnp.minimum(iota + 1, 15)

    def issue(i, boff, sem):
        bu = a0 + i * C
        base = pl.multiple_of(jnp.minimum(bu, NPINS - C), 16)
        pltpu.make_async_copy(
            p2n_h.at[pl.ds(base, C)], nbuf.at[pl.ds(boff, C)], sem).start()
        pltpu.make_async_copy(
            pos_h.at[pl.ds(base, C)], xbuf.at[pl.ds(boff, C)], sem).start()
        pltpu.make_async_copy(
            pos_h.at[pl.ds(base + NPINS, C)], ybuf.at[pl.ds(boff, C)],
            sem).start()

    def drain(boff, sem):
        pltpu.make_async_copy(
            p2n_h.at[pl.ds(0, C)], nbuf.at[pl.ds(boff, C)], sem).wait()
        pltpu.make_async_copy(
            pos_h.at[pl.ds(0, C)], xbuf.at[pl.ds(boff, C)], sem).wait()
        pltpu.make_async_copy(
            pos_h.at[pl.ds(0, C)], ybuf.at[pl.ds(boff, C)], sem).wait()

    def do_pass(grp_fn):
        def proc(i, boff):
            bu = a0 + i * C
            base = pl.multiple_of(jnp.minimum(bu, NPINS - C), 16)
            lo = jnp.maximum(p0, bu)

            def grp(j, carry2):
                boff16 = pl.multiple_of(boff + j * 16, 16)
                nv = nbuf[pl.ds(boff16, 16)]
                xv = xbuf[pl.ds(boff16, 16)]
                yv = ybuf[pl.ds(boff16, 16)]
                g = base + j * 16 + iota
                mval = (g >= lo) & (g < p1)
                nloc = jnp.where(mval, nv - n0, DUMMY)
                grp_fn(xv, yv, mval, nloc)
                return carry2

            lax.fori_loop(0, C // 16, grp, 0)

        pl.when(nch > 0)(lambda: issue(0, 0, sem_a))

        def body2(i2, carry):
            i = i2 * 2
            pl.when(i + 1 < nch)(lambda: issue(i + 1, C, sem_b))
            drain(0, sem_a)
            proc(i, 0)

            @pl.when(i + 1 < nch)
            def _():
                pl.when(i + 2 < nch)(lambda: issue(i + 2, 0, sem_a))
                drain(C, sem_b)
                proc(i + 1, C)

            return carry

        lax.fori_loop(0, (nch + 1) // 2, body2, 0)

    def seg_masks(nloc):
        sks = tuple((iota >= k) & (_take(nloc, idxk[kk]) == nloc)
                    for kk, k in enumerate((1, 2, 4, 8)))
        nnxt = _take(nloc, idxn)
        isl = (iota == 15) | (nloc != nnxt)
        return sks, isl

    def grp1(xv, yv, mval, nloc):
        sks, isl = seg_masks(nloc)
        lm = isl & mval

        def segscan(v, op):
            m = v
            for kk in range(4):
                m = jnp.where(sks[kk], op(_take(m, idxk[kk]), m), m)
            return m

        cur = plsc.load_gather(mxx, [nloc])
        plsc.store_scatter(mxx, [nloc],
                           jnp.maximum(cur, segscan(xv, jnp.maximum)), mask=lm)
        cur = plsc.load_gather(mnx, [nloc])
        plsc.store_scatter(mnx, [nloc],
                           jnp.minimum(cur, segscan(xv, jnp.minimum)), mask=lm)
        cur = plsc.load_gather(mxy, [nloc])
        plsc.store_scatter(mxy, [nloc],
                           jnp.maximum(cur, segscan(yv, jnp.maximum)), mask=lm)
        cur = plsc.load_gather(mny, [nloc])
        plsc.store_scatter(mny, [nloc],
                           jnp.minimum(cur, segscan(yv, jnp.minimum)), mask=lm)

    def grp2(xv, yv, mval, nloc):
        sks, isl = seg_masks(nloc)
        lm = isl & mval
        mpx = plsc.load_gather(mxx, [nloc])
        mnx_ = plsc.load_gather(mnx, [nloc])
        mpy = plsc.load_gather(mxy, [nloc])
        mny_ = plsc.load_gather(mny, [nloc])
        epx = jnp.where(mval, jnp.exp((xv - mpx) * gv), 0.0)
        enx = jnp.where(mval, jnp.exp((mnx_ - xv) * gv), 0.0)
        epy = jnp.where(mval, jnp.exp((yv - mpy) * gv), 0.0)
        eny = jnp.where(mval, jnp.exp((mny_ - yv) * gv), 0.0)

        def runsum(q):
            m = q
            for kk in range(4):
                m = jnp.where(sks[kk], _take(m, idxk[kk]) + m, m)
            return m

        plsc.addupdate_scatter(spx, [nloc], runsum(epx), mask=lm)
        plsc.addupdate_scatter(sxpx, [nloc], runsum(xv * epx), mask=lm)
        plsc.addupdate_scatter(snx, [nloc], runsum(enx), mask=lm)
        plsc.addupdate_scatter(sxnx, [nloc], runsum(xv * enx), mask=lm)
        plsc.addupdate_scatter(spy, [nloc], runsum(epy), mask=lm)
        plsc.addupdate_scatter(sxpy, [nloc], runsum(yv * epy), mask=lm)
        plsc.addupdate_scatter(sny, [nloc], runsum(eny), mask=lm)
        plsc.addupdate_scatter(sxny, [nloc], runsum(yv * eny), mask=lm)

    do_pass(grp1)
    do_pass(grp2)

    one = jnp.ones((16,), jnp.float32)

    def net_grp(j, carry):
        dsl = pl.ds(pl.multiple_of(j * 16, 16), 16)
        ok = (j * 16 + iota) < ncnt
        sp = spx[dsl]
        has = ok & (sp > 0.0)
        spd = jnp.where(has, sp, one)
        snd = jnp.where(has, snx[dsl], one)
        wlx = sxpx[dsl] / spd - sxnx[dsl] / snd
        spd2 = jnp.where(has, spy[dsl], one)
        snd2 = jnp.where(has, sny[dsl], one)
        wly = sxpy[dsl] / spd2 - sxny[dsl] / snd2
        contrib = jnp.where(has, wx_v[dsl] * wlx + wy_v[dsl] * wly, 0.0)
        accv[...] = accv[...] + contrib
        return carry

    lax.fori_loop(0, NPW // 16, net_grp, 0)
    pltpu.sync_copy(accv, out_h.at[wid])


@jax.jit
def _wirelength_sc(pos, p2n, bounds, wx, wy, ginv):
    mesh = plsc.VectorSubcoreMesh(core_axis_name="c", subcore_axis_name="s")
    f = functools.partial(
        pl.kernel,
        mesh=mesh,
        compiler_params=pltpu.CompilerParams(needs_layout_passes=False),
        out_type=jax.ShapeDtypeStruct((NW, 16), jnp.float32),
        scratch_types=[
            pltpu.VMEM((48,), jnp.int32),
            pltpu.VMEM((16,), jnp.float32),
            pltpu.VMEM((NPW,), jnp.float32),
            pltpu.VMEM((NPW,), jnp.float32),
            pltpu.VMEM((2 * C,), jnp.int32),
            pltpu.VMEM((2 * C,), jnp.float32),
            pltpu.VMEM((2 * C,), jnp.float32),
        ] + [pltpu.VMEM((NLOC,), jnp.float32)] * 12 + [
            pltpu.VMEM((16,), jnp.float32),
            pltpu.SemaphoreType.DMA,
            pltpu.SemaphoreType.DMA,
        ],
    )(_sc_body)
    return f(pos, p2n, bounds, wx, wy, ginv)


def kernel(pos, pin2net_map, flat_netpin, netpin_start, net_weights,
           net_weights_x, net_mask, pin_mask, inv_gamma):
    maskf = net_mask.astype(jnp.float32)
    wx = jnp.pad(net_weights_x * maskf, (0, NPAD - NNETS))
    wy = jnp.pad(net_weights * maskf, (0, NPAD - NNETS))
    bidx = jnp.minimum(jnp.arange(33, dtype=jnp.int32) * NPW, NNETS)
    bounds = jnp.pad(netpin_start[bidx], (0, 48 - 33), mode="edge")
    ginv = jnp.broadcast_to(inv_gamma.astype(jnp.float32), (16,))
    out = _wirelength_sc(pos, pin2net_map, bounds, wx, wy, ginv)
    return jnp.sum(out)


# speedup vs baseline: 2.1818x; 1.0040x over previous
import functools

import jax
import jax.numpy as jnp
from jax import lax
from jax.experimental import pallas as pl
from jax.experimental.pallas import tpu as pltpu
from jax.experimental.pallas import tpu_sc as plsc

NPINS = 1_600_000
NNETS = 100_000
NC = 2
NS = 16
NW = NC * NS
NPW = 3136
NLOC = 3152
DUMMY = NLOC - 1
C = 8192
NPAD = NW * NPW


def _take(v, idx):
    return v.at[idx].get(mode="promise_in_bounds")


def _extract(vmem, j, iota):
    del iota
    base = pl.multiple_of((j // 16) * 16, 16)
    v = vmem[pl.ds(base, 16)]
    sel = _take(v, jnp.broadcast_to(j - base, (16,)))
    return sel[0]


def _sc_body(pos_h, p2n_h, bnd_h, wx_h, wy_h, gi_h, out_h,
             bnds_v, gi_v, wx_v, wy_v, nbuf, xbuf, ybuf,
             mxx, mnx, mxy, mny,
             spx, sxpx, snx, sxnx, spy, sxpy, sny, sxny, accv, sem_a, sem_b):
    iota = lax.broadcasted_iota(jnp.int32, (16,), 0)
    c = lax.axis_index("c")
    s = lax.axis_index("s")
    wid = s * NC + c
    n0 = pl.multiple_of(wid * NPW, 16)
    ncnt = jnp.minimum(NNETS - n0, NPW)

    pltpu.sync_copy(bnd_h, bnds_v)
    pltpu.sync_copy(gi_h, gi_v)
    pltpu.sync_copy(wx_h.at[pl.ds(n0, NPW)], wx_v)
    pltpu.sync_copy(wy_h.at[pl.ds(n0, NPW)], wy_v)

    p0 = _extract(bnds_v, wid, iota)
    p1 = _extract(bnds_v, wid + 1, iota)
    a0 = pl.multiple_of((p0 // 16) * 16, 16)
    nch = (p1 - a0 + C - 1) // C

    gv = gi_v[...]

    big = jnp.full((16,), 3.0e38, jnp.float32)
    zf = jnp.zeros((16,), jnp.float32)

    def initb(j, carry):
        dsl = pl.ds(pl.multiple_of(j * 16, 16), 16)
        mxx[dsl] = -big
        mnx[dsl] = big
        mxy[dsl] = -big
        mny[dsl] = big
        spx[dsl] = zf
        sxpx[dsl] = zf
        snx[dsl] = zf
        sxnx[dsl] = zf
        spy[dsl] = zf
        sxpy[dsl] = zf
        sny[dsl] = zf
        sxny[dsl] = zf
        return carry

    lax.fori_loop(0, NLOC // 16, initb, 0)
    accv[...] = zf

    idxk = tuple(jnp.maximum(iota - k, 0) for k in (1, 2, 4, 8))
    idxn = jnp.minimum(iota + 1, 15)

    def issue(i, boff, sem):
        bu = a0 + i * C
        base = pl.multiple_of(jnp.minimum(bu, NPINS - C), 16)
        pltpu.make_async_copy(
            p2n_h.at[pl.ds(base, C)], nbuf.at[pl.ds(boff, C)], sem).start()
        pltpu.make_async_copy(
            pos_h.at[pl.ds(base, C)], xbuf.at[pl.ds(boff, C)], sem).start()
        pltpu.make_async_copy(
            pos_h.at[pl.ds(base + NPINS, C)], ybuf.at[pl.ds(boff, C)],
            sem).start()

    def drain(boff, sem):
        pltpu.make_async_copy(
            p2n_h.at[pl.ds(0, C)], nbuf.at[pl.ds(boff, C)], sem).wait()
        pltpu.make_async_copy(
            pos_h.at[pl.ds(0, C)], xbuf.at[pl.ds(boff, C)], sem).wait()
        pltpu.make_async_copy(
            pos_h.at[pl.ds(0, C)], ybuf.at[pl.ds(boff, C)], sem).wait()

    def do_pass(grp_fn):
        def proc(i, boff):
            bu = a0 + i * C
            base = pl.multiple_of(jnp.minimum(bu, NPINS - C), 16)
            lo = jnp.maximum(p0, bu)

            def grp(j, carry2):
                for u in range(4):
                    boff16 = pl.multiple_of(boff + j * 64 + u * 16, 16)
                    nv = nbuf[pl.ds(boff16, 16)]
                    xv = xbuf[pl.ds(boff16, 16)]
                    yv = ybuf[pl.ds(boff16, 16)]
                    g = base + j * 64 + u * 16 + iota
                    mval = (g >= lo) & (g < p1)
                    nloc = jnp.where(mval, nv - n0, DUMMY)
                    grp_fn(xv, yv, mval, nloc)
                return carry2

            lax.fori_loop(0, C // 64, grp, 0)

        pl.when(nch > 0)(lambda: issue(0, 0, sem_a))

        def body2(i2, carry):
            i = i2 * 2
            pl.when(i + 1 < nch)(lambda: issue(i + 1, C, sem_b))
            drain(0, sem_a)
            proc(i, 0)

            @pl.when(i + 1 < nch)
            def _():
                pl.when(i + 2 < nch)(lambda: issue(i + 2, 0, sem_a))
                drain(C, sem_b)
                proc(i + 1, C)

            return carry

        lax.fori_loop(0, (nch + 1) // 2, body2, 0)

    def seg_masks(nloc):
        sks = tuple((iota >= k) & (_take(nloc, idxk[kk]) == nloc)
                    for kk, k in enumerate((1, 2, 4, 8)))
        nnxt = _take(nloc, idxn)
        isl = (iota == 15) | (nloc != nnxt)
        return sks, isl

    def grp1(xv, yv, mval, nloc):
        sks, isl = seg_masks(nloc)
        lm = isl & mval

        def segscan(v, op):
            m = v
            for kk in range(4):
                m = jnp.where(sks[kk], op(_take(m, idxk[kk]), m), m)
            return m

        cur = plsc.load_gather(mxx, [nloc])
        plsc.store_scatter(mxx, [nloc],
                           jnp.maximum(cur, segscan(xv, jnp.maximum)), mask=lm)
        cur = plsc.load_gather(mnx, [nloc])
        plsc.store_scatter(mnx, [nloc],
                           jnp.minimum(cur, segscan(xv, jnp.minimum)), mask=lm)
        cur = plsc.load_gather(mxy, [nloc])
        plsc.store_scatter(mxy, [nloc],
                           jnp.maximum(cur, segscan(yv, jnp.maximum)), mask=lm)
        cur = plsc.load_gather(mny, [nloc])
        plsc.store_scatter(mny, [nloc],
                           jnp.minimum(cur, segscan(yv, jnp.minimum)), mask=lm)

    def grp2(xv, yv, mval, nloc):
        sks, isl = seg_masks(nloc)
        lm = isl & mval
        mpx = plsc.load_gather(mxx, [nloc])
        mnx_ = plsc.load_gather(mnx, [nloc])
        mpy = plsc.load_gather(mxy, [nloc])
        mny_ = plsc.load_gather(mny, [nloc])
        epx = jnp.where(mval, jnp.exp((xv - mpx) * gv), 0.0)
        enx = jnp.where(mval, jnp.exp((mnx_ - xv) * gv), 0.0)
        epy = jnp.where(mval, jnp.exp((yv - mpy) * gv), 0.0)
        eny = jnp.where(mval, jnp.exp((mny_ - yv) * gv), 0.0)

        def runsum(q):
            m = q
            for kk in range(4):
                m = jnp.where(sks[kk], _take(m, idxk[kk]) + m, m)
            return m

        plsc.addupdate_scatter(spx, [nloc], runsum(epx), mask=lm)
        plsc.addupdate_scatter(sxpx, [nloc], runsum(xv * epx), mask=lm)
        plsc.addupdate_scatter(snx, [nloc], runsum(enx), mask=lm)
        plsc.addupdate_scatter(sxnx, [nloc], runsum(xv * enx), mask=lm)
        plsc.addupdate_scatter(spy, [nloc], runsum(epy), mask=lm)
        plsc.addupdate_scatter(sxpy, [nloc], runsum(yv * epy), mask=lm)
        plsc.addupdate_scatter(sny, [nloc], runsum(eny), mask=lm)
        plsc.addupdate_scatter(sxny, [nloc], runsum(yv * eny), mask=lm)

    do_pass(grp1)
    do_pass(grp2)

    one = jnp.ones((16,), jnp.float32)

    def net_grp(j, carry):
        dsl = pl.ds(pl.multiple_of(j * 16, 16), 16)
        ok = (j * 16 + iota) < ncnt
        sp = spx[dsl]
        has = ok & (sp > 0.0)
        spd = jnp.where(has, sp, one)
        snd = jnp.where(has, snx[dsl], one)
        wlx = sxpx[dsl] / spd - sxnx[dsl] / snd
        spd2 = jnp.where(has, spy[dsl], one)
        snd2 = jnp.where(has, sny[dsl], one)
        wly = sxpy[dsl] / spd2 - sxny[dsl] / snd2
        contrib = jnp.where(has, wx_v[dsl] * wlx + wy_v[dsl] * wly, 0.0)
        accv[...] = accv[...] + contrib
        return carry

    lax.fori_loop(0, NPW // 16, net_grp, 0)
    pltpu.sync_copy(accv, out_h.at[wid])


@jax.jit
def _wirelength_sc(pos, p2n, bounds, wx, wy, ginv):
    mesh = plsc.VectorSubcoreMesh(core_axis_name="c", subcore_axis_name="s")
    f = functools.partial(
        pl.kernel,
        mesh=mesh,
        compiler_params=pltpu.CompilerParams(needs_layout_passes=False),
        out_type=jax.ShapeDtypeStruct((NW, 16), jnp.float32),
        scratch_types=[
            pltpu.VMEM((48,), jnp.int32),
            pltpu.VMEM((16,), jnp.float32),
            pltpu.VMEM((NPW,), jnp.float32),
            pltpu.VMEM((NPW,), jnp.float32),
            pltpu.VMEM((2 * C,), jnp.int32),
            pltpu.VMEM((2 * C,), jnp.float32),
            pltpu.VMEM((2 * C,), jnp.float32),
        ] + [pltpu.VMEM((NLOC,), jnp.float32)] * 12 + [
            pltpu.VMEM((16,), jnp.float32),
            pltpu.SemaphoreType.DMA,
            pltpu.SemaphoreType.DMA,
        ],
    )(_sc_body)
    return f(pos, p2n, bounds, wx, wy, ginv)


def kernel(pos, pin2net_map, flat_netpin, netpin_start, net_weights,
           net_weights_x, net_mask, pin_mask, inv_gamma):
    maskf = net_mask.astype(jnp.float32)
    wx = jnp.pad(net_weights_x * maskf, (0, NPAD - NNETS))
    wy = jnp.pad(net_weights * maskf, (0, NPAD - NNETS))
    bidx = jnp.minimum(jnp.arange(33, dtype=jnp.int32) * NPW, NNETS)
    bounds = jnp.pad(netpin_start[bidx], (0, 48 - 33), mode="edge")
    ginv = jnp.broadcast_to(inv_gamma.astype(jnp.float32), (16,))
    out = _wirelength_sc(pos, pin2net_map, bounds, wx, wy, ginv)
    return jnp.sum(out)
